# baseline (speedup 1.0000x reference)
import functools

import jax
import jax.numpy as jnp
from jax import lax
from jax.experimental import pallas as pl
from jax.experimental.pallas import tpu as pltpu

B, H, W, D = 16, 24, 24, 256
K = 1024
HW = H * W
COMMIT = 0.25
GROUP = 8
ROWS = GROUP * HW
STEPS = B // GROUP
NTOT = B * HW * D


def _vq_fast_body(z_ref, emb_ref, q_ref, idx_ref, loss_ref, bad_ref,
                  en_ref, rhs_ref):
    g = pl.program_id(0)
    emb = emb_ref[...]

    @pl.when(g == 0)
    def _():
        en_ref[...] = jnp.sum(emb * emb, axis=1, keepdims=True)
        ki = lax.broadcasted_iota(jnp.int32, (8, K), 1)
        rowi = lax.broadcasted_iota(jnp.int32, (8, K), 0)
        rhs_ref[...] = jnp.where(
            rowi == 0, (ki // 32).astype(jnp.float32),
            jnp.where(rowi == 1, (ki % 32).astype(jnp.float32),
                      jnp.where(rowi == 2, 1.0, 0.0)))
        loss_ref[0, 0] = 0.0
        bad_ref[0, 0] = 0

    bad = jnp.int32(0)
    part = jnp.float32(0.0)
    for j in range(GROUP):
        zj = z_ref[0, j * HW:(j + 1) * HW, :]
        zn = jnp.sum(zj * zj, axis=1, keepdims=True)
        znt = lax.transpose(zn, (1, 0))
        zj2t = lax.transpose(zj * (-2.0), (1, 0))
        s2t = lax.dot_general(emb, zj2t, (((1,), (0,)), ((), ())),
                              preferred_element_type=jnp.float32)
        d2t = (znt + s2t) + en_ref[...]

        m_d2 = jnp.min(d2t, axis=0, keepdims=True)
        c = jnp.sqrt(jnp.maximum(m_d2, 0.0))
        mb = lax.bitcast_convert_type(m_d2, jnp.int32)
        cb = lax.bitcast_convert_type(c, jnp.int32)
        ext = jnp.zeros(mb.shape, jnp.int32)
        for i in range(1, 5):
            xi = lax.bitcast_convert_type(mb + i, jnp.float32)
            si = jnp.sqrt(jnp.maximum(xi, 0.0))
            ext += (lax.bitcast_convert_type(si, jnp.int32) == cb
                    ).astype(jnp.int32)
        hi = lax.bitcast_convert_type(mb + ext, jnp.float32)
        hi = jnp.where(c == 0.0, 0.0, hi)

        loose = (d2t <= hi).astype(jnp.float32)
        t = lax.dot_general(rhs_ref[...], loose, (((1,), (0,)), ((), ())),
                            preferred_element_type=jnp.float32)
        idx_f = 32.0 * t[0:1, :] + t[1:2, :]
        cnt = t[2:3, :]
        bad = bad | jnp.any(cnt != 1.0).astype(jnp.int32)

        idx_ref[j, :, :] = idx_f.astype(jnp.int32)
        q_ref[j] = lax.dot_general(emb, loose, (((0,), (0,)), ((), ())),
                                   preferred_element_type=jnp.float32)
        part += jnp.sum(jnp.maximum(m_d2, 0.0))

    loss_ref[0, 0] += part
    bad_ref[0, 0] |= bad

    @pl.when(g == STEPS - 1)
    def _():
        loss_ref[0, 0] = loss_ref[0, 0] / jnp.float32(NTOT) * jnp.float32(COMMIT)


def _vq_exact_body(z_ref, emb_ref, q_ref, idx_ref, loss_ref, en_ref):
    g = pl.program_id(0)
    z = z_ref[0]
    emb = emb_ref[...]

    @pl.when(g == 0)
    def _():
        en_ref[...] = jnp.sum(emb * emb, axis=1)[None, :]
        loss_ref[0, 0] = 0.0

    zn = jnp.sum(z * z, axis=1, keepdims=True)
    s2 = lax.dot_general(z * (-2.0), emb, (((1,), (1,)), ((), ())),
                         preferred_element_type=jnp.float32)
    dist = jnp.sqrt(jnp.maximum((zn + s2) + en_ref[...], 0.0))
    m = jnp.min(dist, axis=1, keepdims=True)
    kiota = lax.broadcasted_iota(jnp.int32, (ROWS, K), 1)
    idx = jnp.min(jnp.where(dist == m, kiota, K), axis=1)
    kiota_hw = lax.broadcasted_iota(jnp.int32, (HW, K), 1)
    for j in range(GROUP):
        idx_j = idx[j * HW:(j + 1) * HW]
        oh = (kiota_hw == idx_j[:, None]).astype(jnp.float32)
        q_ref[j] = lax.dot_general(emb, oh, (((0,), (1,)), ((), ())),
                                   preferred_element_type=jnp.float32)
        idx_ref[j, 0, :] = idx_j
    loss_ref[0, 0] += jnp.sum(m * m)

    @pl.when(g == STEPS - 1)
    def _():
        loss_ref[0, 0] = loss_ref[0, 0] / jnp.float32(NTOT) * jnp.float32(COMMIT)


_OUT_SPECS = [
    pl.BlockSpec((GROUP, D, HW), lambda g: (g, 0, 0)),
    pl.BlockSpec((GROUP, 1, HW), lambda g: (g, 0, 0)),
    pl.BlockSpec((1, 1), lambda g: (0, 0), memory_space=pltpu.SMEM),
]
_OUT_SHAPE = [
    jax.ShapeDtypeStruct((B, D, HW), jnp.float32),
    jax.ShapeDtypeStruct((B, 1, HW), jnp.int32),
    jax.ShapeDtypeStruct((1, 1), jnp.float32),
]
_IN_SPECS = [
    pl.BlockSpec((1, ROWS, D), lambda g: (g, 0, 0)),
    pl.BlockSpec((K, D), lambda g: (0, 0)),
]


def _run_fast(z3, embeddings, interpret):
    return pl.pallas_call(
        _vq_fast_body,
        grid=(STEPS,),
        in_specs=_IN_SPECS,
        out_specs=_OUT_SPECS + [
            pl.BlockSpec((1, 1), lambda g: (0, 0), memory_space=pltpu.SMEM)],
        out_shape=_OUT_SHAPE + [jax.ShapeDtypeStruct((1, 1), jnp.int32)],
        scratch_shapes=[pltpu.VMEM((K, 1), jnp.float32),
                        pltpu.VMEM((8, K), jnp.float32)],
        interpret=interpret,
    )(z3, embeddings)


def _run_exact(z3, embeddings, interpret):
    return pl.pallas_call(
        _vq_exact_body,
        grid=(STEPS,),
        in_specs=_IN_SPECS,
        out_specs=_OUT_SPECS,
        out_shape=_OUT_SHAPE,
        scratch_shapes=[pltpu.VMEM((1, K), jnp.float32)],
        interpret=interpret,
    )(z3, embeddings)


@functools.partial(jax.jit, static_argnames=("interpret",))
def _vq(z, embeddings, interpret=False):
    z3 = z.reshape(STEPS, ROWS, D)
    q, idx, loss_sum, bad = _run_fast(z3, embeddings, interpret)
    q, idx, loss_sum = lax.cond(
        bad[0, 0] != 0,
        lambda ops: _run_exact(ops[0], ops[1], interpret),
        lambda ops: (ops[2], ops[3], ops[4]),
        (z3, embeddings, q, idx, loss_sum),
    )
    quantized_out = q.reshape(B, D, H, W)
    indices = idx.reshape(B, 1, H, W)
    return quantized_out, indices, loss_sum.reshape(())


def kernel(z, embeddings):
    return _vq(z, embeddings)


# speedup vs baseline: 1.0076x; 1.0076x over previous
import functools

import jax
import jax.numpy as jnp
from jax import lax
from jax.experimental import pallas as pl
from jax.experimental.pallas import tpu as pltpu

B, H, W, D = 16, 24, 24, 256
K = 1024
HW = H * W
COMMIT = 0.25
GROUP = 4
ROWS = GROUP * HW
STEPS = B // GROUP
NTOT = B * HW * D


def _vq_fast_body(z_ref, emb_ref, q_ref, idx_ref, loss_ref, bad_ref,
                  en_ref, rhs_ref):
    g = pl.program_id(0)
    emb = emb_ref[...]

    @pl.when(g == 0)
    def _():
        en_ref[...] = jnp.sum(emb * emb, axis=1, keepdims=True)
        ki = lax.broadcasted_iota(jnp.int32, (8, K), 1)
        rowi = lax.broadcasted_iota(jnp.int32, (8, K), 0)
        rhs_ref[...] = jnp.where(
            rowi == 0, (ki // 32).astype(jnp.float32),
            jnp.where(rowi == 1, (ki % 32).astype(jnp.float32),
                      jnp.where(rowi == 2, 1.0, 0.0)))
        loss_ref[0, 0] = 0.0
        bad_ref[0, 0] = 0

    bad = jnp.int32(0)
    part = jnp.float32(0.0)
    for j in range(GROUP):
        zj = z_ref[0, j * HW:(j + 1) * HW, :]
        zn = jnp.sum(zj * zj, axis=1, keepdims=True)
        znt = lax.transpose(zn, (1, 0))
        zj2t = lax.transpose(zj * (-2.0), (1, 0))
        s2t = lax.dot_general(emb, zj2t, (((1,), (0,)), ((), ())),
                              preferred_element_type=jnp.float32)
        d2t = (znt + s2t) + en_ref[...]

        m_d2 = jnp.min(d2t, axis=0, keepdims=True)
        c = jnp.sqrt(jnp.maximum(m_d2, 0.0))
        mb = lax.bitcast_convert_type(m_d2, jnp.int32)
        cb = lax.bitcast_convert_type(c, jnp.int32)
        ext = jnp.zeros(mb.shape, jnp.int32)
        for i in range(1, 5):
            xi = lax.bitcast_convert_type(mb + i, jnp.float32)
            si = jnp.sqrt(jnp.maximum(xi, 0.0))
            ext += (lax.bitcast_convert_type(si, jnp.int32) == cb
                    ).astype(jnp.int32)
        hi = lax.bitcast_convert_type(mb + ext, jnp.float32)
        hi = jnp.where(c == 0.0, 0.0, hi)

        loose = (d2t <= hi).astype(jnp.float32)
        t = lax.dot_general(rhs_ref[...], loose, (((1,), (0,)), ((), ())),
                            preferred_element_type=jnp.float32)
        idx_f = 32.0 * t[0:1, :] + t[1:2, :]
        cnt = t[2:3, :]
        bad = bad | jnp.any(cnt != 1.0).astype(jnp.int32)

        idx_ref[j, :, :] = idx_f.astype(jnp.int32)
        q_ref[j] = lax.dot_general(emb, loose, (((0,), (0,)), ((), ())),
                                   preferred_element_type=jnp.float32)
        part += jnp.sum(jnp.maximum(m_d2, 0.0))

    loss_ref[0, 0] += part
    bad_ref[0, 0] |= bad

    @pl.when(g == STEPS - 1)
    def _():
        loss_ref[0, 0] = loss_ref[0, 0] / jnp.float32(NTOT) * jnp.float32(COMMIT)


def _vq_exact_body(z_ref, emb_ref, q_ref, idx_ref, loss_ref, en_ref):
    g = pl.program_id(0)
    z = z_ref[0]
    emb = emb_ref[...]

    @pl.when(g == 0)
    def _():
        en_ref[...] = jnp.sum(emb * emb, axis=1)[None, :]
        loss_ref[0, 0] = 0.0

    zn = jnp.sum(z * z, axis=1, keepdims=True)
    s2 = lax.dot_general(z * (-2.0), emb, (((1,), (1,)), ((), ())),
                         preferred_element_type=jnp.float32)
    dist = jnp.sqrt(jnp.maximum((zn + s2) + en_ref[...], 0.0))
    m = jnp.min(dist, axis=1, keepdims=True)
    kiota = lax.broadcasted_iota(jnp.int32, (ROWS, K), 1)
    idx = jnp.min(jnp.where(dist == m, kiota, K), axis=1)
    kiota_hw = lax.broadcasted_iota(jnp.int32, (HW, K), 1)
    for j in range(GROUP):
        idx_j = idx[j * HW:(j + 1) * HW]
        oh = (kiota_hw == idx_j[:, None]).astype(jnp.float32)
        q_ref[j] = lax.dot_general(emb, oh, (((0,), (1,)), ((), ())),
                                   preferred_element_type=jnp.float32)
        idx_ref[j, 0, :] = idx_j
    loss_ref[0, 0] += jnp.sum(m * m)

    @pl.when(g == STEPS - 1)
    def _():
        loss_ref[0, 0] = loss_ref[0, 0] / jnp.float32(NTOT) * jnp.float32(COMMIT)


_OUT_SPECS = [
    pl.BlockSpec((GROUP, D, HW), lambda g: (g, 0, 0)),
    pl.BlockSpec((GROUP, 1, HW), lambda g: (g, 0, 0)),
    pl.BlockSpec((1, 1), lambda g: (0, 0), memory_space=pltpu.SMEM),
]
_OUT_SHAPE = [
    jax.ShapeDtypeStruct((B, D, HW), jnp.float32),
    jax.ShapeDtypeStruct((B, 1, HW), jnp.int32),
    jax.ShapeDtypeStruct((1, 1), jnp.float32),
]
_IN_SPECS = [
    pl.BlockSpec((1, ROWS, D), lambda g: (g, 0, 0)),
    pl.BlockSpec((K, D), lambda g: (0, 0)),
]


def _run_fast(z3, embeddings, interpret):
    return pl.pallas_call(
        _vq_fast_body,
        grid=(STEPS,),
        in_specs=_IN_SPECS,
        out_specs=_OUT_SPECS + [
            pl.BlockSpec((1, 1), lambda g: (0, 0), memory_space=pltpu.SMEM)],
        out_shape=_OUT_SHAPE + [jax.ShapeDtypeStruct((1, 1), jnp.int32)],
        scratch_shapes=[pltpu.VMEM((K, 1), jnp.float32),
                        pltpu.VMEM((8, K), jnp.float32)],
        interpret=interpret,
    )(z3, embeddings)


def _run_exact(z3, embeddings, interpret):
    return pl.pallas_call(
        _vq_exact_body,
        grid=(STEPS,),
        in_specs=_IN_SPECS,
        out_specs=_OUT_SPECS,
        out_shape=_OUT_SHAPE,
        scratch_shapes=[pltpu.VMEM((1, K), jnp.float32)],
        interpret=interpret,
    )(z3, embeddings)


@functools.partial(jax.jit, static_argnames=("interpret",))
def _vq(z, embeddings, interpret=False):
    z3 = z.reshape(STEPS, ROWS, D)
    q, idx, loss_sum, bad = _run_fast(z3, embeddings, interpret)
    q, idx, loss_sum = lax.cond(
        bad[0, 0] != 0,
        lambda ops: _run_exact(ops[0], ops[1], interpret),
        lambda ops: (ops[2], ops[3], ops[4]),
        (z3, embeddings, q, idx, loss_sum),
    )
    quantized_out = q.reshape(B, D, H, W)
    indices = idx.reshape(B, 1, H, W)
    return quantized_out, indices, loss_sum.reshape(())


def kernel(z, embeddings):
    return _vq(z, embeddings)
